# baseline (speedup 1.0000x reference)
import functools

import jax
import jax.numpy as jnp
from jax import lax
from jax.experimental import pallas as pl
from jax.experimental.pallas import tpu as pltpu
from jax.experimental.pallas import tpu_sc as plsc

_NCORES = 2
_NSUB = 16
_NW = _NCORES * _NSUB
_L = 16


def kernel(kspace_data, mask):
    nb, nc, nx, ny = kspace_data.shape
    n_sampled = mask.shape[2] // 2
    nrows = nb * nc * nx
    rows_per_w = nrows // _NW
    planes_per_w = rows_per_w // nx
    xblk = 128
    sub_rows = 32
    nsub = xblk // sub_rows
    nblks = rows_per_w // xblk
    blks_per_plane = nx // xblk
    ngroups = n_sampled // _L

    x2 = kspace_data.reshape(nrows, ny)
    mask_i32 = mask.reshape(ny).astype(jnp.int32)

    mesh = plsc.VectorSubcoreMesh(
        core_axis_name="c", subcore_axis_name="s",
        num_cores=_NCORES, num_subcores=_NSUB)

    @functools.partial(
        pl.kernel,
        out_type=jax.ShapeDtypeStruct((nb, nc, n_sampled, nx), jnp.float32),
        mesh=mesh,
        scratch_types=[
            pltpu.VMEM((sub_rows, ny), jnp.float32),
            pltpu.VMEM((sub_rows, ny), jnp.float32),
            pltpu.VMEM((n_sampled, xblk), jnp.float32),
            pltpu.VMEM((n_sampled, xblk), jnp.float32),
            pltpu.VMEM((ny,), jnp.int32),
            pltpu.VMEM((ny + _L,), jnp.int32),
            pltpu.VMEM((_L,), jnp.int32),
            pltpu.SemaphoreType.DMA,
            pltpu.SemaphoreType.DMA,
            pltpu.SemaphoreType.DMA,
            pltpu.SemaphoreType.DMA,
        ],
        compiler_params=pltpu.CompilerParams(needs_layout_passes=False),
    )
    def run(x_hbm, mask_hbm, out_hbm, in_v0, in_v1, out_v0, out_v1,
            mask_v, cols_v, tmp_v, isem0, isem1, osem0, osem1):
        in_bufs = (in_v0, in_v1)
        out_bufs = (out_v0, out_v1)
        isems = (isem0, isem1)
        osems = (osem0, osem1)

        wid = lax.axis_index("s") * _NCORES + lax.axis_index("c")
        pltpu.sync_copy(mask_hbm, mask_v)

        zero = jnp.zeros((_L,), jnp.int32)
        for g in range(ny // _L + 1):
            cols_v[pl.ds(g * _L, _L)] = zero

        lane = lax.iota(jnp.int32, _L)

        def comp_body(g, off):
            m = mask_v[pl.ds(g * _L, _L)] != 0
            s = jnp.where(m, jnp.int32(1), jnp.int32(0))
            for sh in (1, 2, 4, 8):
                tmp_v[...] = s
                sv = plsc.load_gather(tmp_v, [jnp.maximum(lane - sh, 0)])
                s = s + jnp.where(lane >= sh, sv, jnp.int32(0))
            tmp_v[...] = s
            tot = plsc.load_gather(tmp_v, [jnp.full((_L,), _L - 1,
                                                    jnp.int32)])
            ids = lane + g * _L
            plsc.store_scatter(cols_v, [off + s - 1], ids, mask=m)
            return off + tot
        lax.fori_loop(0, ny // _L, comp_body, jnp.zeros((_L,), jnp.int32))

        cols = [cols_v[pl.ds(g * _L, _L)] for g in range(ngroups)]
        jvs = [lane + g * _L for g in range(ngroups)]

        row0 = wid * rows_per_w

        def in_slice(t):
            return x_hbm.at[pl.ds(row0 + t * sub_rows, sub_rows), :]

        def out_slice(u):
            plane = planes_per_w * wid + u // blks_per_plane
            x0 = (u % blks_per_plane) * xblk
            return out_hbm.at[plane // nc, plane % nc, :, pl.ds(x0, xblk)]

        pltpu.async_copy(in_slice(0), in_bufs[0], isems[0])
        pltpu.async_copy(in_slice(1), in_bufs[1], isems[1])

        def blk_body(u2, _):
            for ub in range(2):
                u = 2 * u2 + ub
                out_b = out_bufs[ub]

                @pl.when(u2 > 0)
                def _():
                    pltpu.make_async_copy(out_b, out_slice(u),
                                          osems[ub]).wait()

                for s in range(nsub):
                    t = u * nsub + s
                    tb = s % 2
                    in_b = in_bufs[tb]
                    pltpu.make_async_copy(in_slice(t), in_b,
                                          isems[tb]).wait()

                    @plsc.parallel_loop(0, sub_rows)
                    def _(r):
                        rv = jnp.zeros((_L,), jnp.int32) + r
                        rr = s * sub_rows + r
                        rrv = jnp.zeros((_L,), jnp.int32) + rr
                        for g in range(ngroups):
                            v = plsc.load_gather(in_b, [rv, cols[g]])
                            plsc.store_scatter(out_b, [jvs[g], rrv], v)

                    @pl.when(t + 2 < nblks * nsub)
                    def _():
                        pltpu.async_copy(in_slice(t + 2), in_b, isems[tb])

                pltpu.async_copy(out_b, out_slice(u), osems[ub])
            return 0
        lax.fori_loop(0, nblks // 2, blk_body, 0)

        for ub in range(2):
            pltpu.make_async_copy(out_bufs[ub],
                                  out_slice(nblks - 2 + ub),
                                  osems[ub]).wait()

    out = run(x2, mask_i32)
    return jnp.transpose(out, (0, 1, 3, 2))
